# baseline (speedup 1.0000x reference)
import jax
import jax.numpy as jnp
from jax.experimental import pallas as pl

E = 64
DIN = 1024
DOUT = 1024
A = 8
R = 8
T = 2048
GS = T // E
AR = A * R
EPB = 2
NB = E // EPB


def _fused_kernel(x_ref, w_ref, a_ref, b_ref, idx_ref, o_ref):
    g = pl.program_id(0)
    col_adapter = jax.lax.broadcasted_iota(jnp.int32, (GS, AR), 1) // R
    for j in range(EPB):
        xs = x_ref[pl.ds((g * EPB + j) * GS, GS), :]
        acc = jnp.dot(xs, w_ref[j], preferred_element_type=jnp.float32)
        inter = jnp.dot(xs.astype(jnp.bfloat16), a_ref[j],
                        preferred_element_type=jnp.float32)
        idxs = idx_ref[0, j * GS:(j + 1) * GS, :]
        mask = (col_adapter == idxs).astype(jnp.float32)
        bmat = b_ref[:, g * EPB + j].reshape(AR, DOUT)
        acc = acc + jnp.dot(inter * mask, bmat, preferred_element_type=jnp.float32)
        o_ref[j * GS:(j + 1) * GS, :] = acc


def kernel(x, group_sizes, adapter_indices_sorted, weight, lora_A, lora_B, lora_scaling):
    a_scaled = lora_A * lora_scaling[:, None, None, None]
    a_stack = a_scaled.transpose(1, 2, 0, 3).reshape(E, DIN, AR).astype(jnp.bfloat16)
    idx = adapter_indices_sorted.reshape(NB, EPB * GS, 1)
    out = pl.pallas_call(
        _fused_kernel,
        grid=(NB,),
        in_specs=[
            pl.BlockSpec((T, DIN), lambda g: (0, 0)),
            pl.BlockSpec((EPB, DIN, DOUT), lambda g: (g, 0, 0)),
            pl.BlockSpec((EPB, DIN, AR), lambda g: (g, 0, 0)),
            pl.BlockSpec((A, E, R, DOUT), lambda g: (0, 0, 0, 0)),
            pl.BlockSpec((1, EPB * GS, 1), lambda g: (g, 0, 0)),
        ],
        out_specs=pl.BlockSpec((EPB * GS, DOUT), lambda g: (g, 0)),
        out_shape=jax.ShapeDtypeStruct((T, DOUT), jnp.float32),
    )(x, weight, a_stack, lora_B, idx)
    return out


# speedup vs baseline: 1.0151x; 1.0151x over previous
import jax
import jax.numpy as jnp
from jax.experimental import pallas as pl

E = 64
DIN = 1024
DOUT = 1024
A = 8
R = 8
T = 2048
GS = T // E
AR = A * R
EPB = 2
NB = E // EPB
KSPLIT = 4
KS = DIN // KSPLIT


def _fused_kernel(x_ref, w0_ref, w1_ref, w2_ref, w3_ref, a_ref, b_ref,
                  idx_ref, o_ref):
    g = pl.program_id(0)
    col_adapter = jax.lax.broadcasted_iota(jnp.int32, (GS, AR), 1) // R
    w_refs = (w0_ref, w1_ref, w2_ref, w3_ref)
    for j in range(EPB):
        xs = x_ref[pl.ds((g * EPB + j) * GS, GS), :]
        acc = jnp.dot(xs[:, 0:KS], w0_ref[j, 0],
                      preferred_element_type=jnp.float32)
        for i in range(1, KSPLIT):
            acc += jnp.dot(xs[:, i * KS:(i + 1) * KS], w_refs[i][j, 0],
                           preferred_element_type=jnp.float32)
        inter = jnp.dot(xs.astype(jnp.bfloat16), a_ref[j],
                        preferred_element_type=jnp.float32)
        idxs = idx_ref[0, j * GS:(j + 1) * GS, :]
        mask = (col_adapter == idxs).astype(jnp.float32)
        bmat = b_ref[:, g * EPB + j].reshape(AR, DOUT)
        acc = acc + jnp.dot(inter * mask, bmat, preferred_element_type=jnp.float32)
        o_ref[j * GS:(j + 1) * GS, :] = acc


def kernel(x, group_sizes, adapter_indices_sorted, weight, lora_A, lora_B, lora_scaling):
    a_scaled = lora_A * lora_scaling[:, None, None, None]
    a_stack = a_scaled.transpose(1, 2, 0, 3).reshape(E, DIN, AR).astype(jnp.bfloat16)
    idx = adapter_indices_sorted.reshape(NB, EPB * GS, 1)
    wr = weight.reshape(E, KSPLIT, KS, DOUT)
    w_specs = [
        pl.BlockSpec((EPB, 1, KS, DOUT), lambda g, i=i: (g, i, 0, 0))
        for i in range(KSPLIT)
    ]
    out = pl.pallas_call(
        _fused_kernel,
        grid=(NB,),
        in_specs=[
            pl.BlockSpec((T, DIN), lambda g: (0, 0)),
            *w_specs,
            pl.BlockSpec((EPB, DIN, AR), lambda g: (g, 0, 0)),
            pl.BlockSpec((A, E, R, DOUT), lambda g: (0, 0, 0, 0)),
            pl.BlockSpec((1, EPB * GS, 1), lambda g: (g, 0, 0)),
        ],
        out_specs=pl.BlockSpec((EPB * GS, DOUT), lambda g: (g, 0)),
        out_shape=jax.ShapeDtypeStruct((T, DOUT), jnp.float32),
    )(x, wr, wr, wr, wr, a_stack, lora_B, idx)
    return out
